# Initial kernel scaffold; baseline (speedup 1.0000x reference)
import jax
import jax.numpy as jnp
from jax.experimental import pallas as pl


def kernel(x_attr, x_stru, ei_attr_pos0, ei_attr_pos1, ei_attr_neg0, ei_attr_neg1, ei_stru_pos0, ei_stru_pos1, ei_stru_neg0, ei_stru_neg1, params_pos, params_neg):
    raise NotImplementedError("write your pallas kernel here")


# speedup vs baseline: 2.2941x; 2.2941x over previous
import functools

import jax
import jax.numpy as jnp
from jax import lax
from jax.experimental import pallas as pl
from jax.experimental.pallas import tpu as pltpu
from jax.experimental.pallas import tpu_sc as plsc

N = 50000
D = 128
E = 50000
BATCH = 128
NB = 400
EP = NB * BATCH
NSUB = 16
NCORE = 2
TPB = NB // NSUB
NPAD = 50176
DSTRIPE = NPAD // NSUB
NCHUNK = 8
CHUNK = NPAD // NCHUNK
CSTRIPE = CHUNK // NSUB
EPT = EP // NSUB
RB = 512
NRB = NPAD // RB


def _sc_mesh():
  return plsc.VectorSubcoreMesh(
      core_axis_name="c", subcore_axis_name="s",
      num_cores=NCORE, num_subcores=NSUB)


def _deg_body(ei_ref, deg_ref, hist, ones_v, idx_v, zrow_v):
  c = lax.axis_index("c")
  s = lax.axis_index("s")
  one16 = jnp.ones((16,), jnp.float32)
  zero16 = jnp.zeros((16,), jnp.float32)
  for i in range(BATCH // 16):
    ones_v[pl.ds(i * 16, 16)] = one16

  def zinit(i, carry):
    zrow_v[pl.ds(i * 16, 16)] = zero16
    return carry
  lax.fori_loop(0, DSTRIPE // 16, zinit, 0)

  for a in range(8):
    pltpu.sync_copy(zrow_v,
                    hist.at[a, pl.ds(s * DSTRIPE, DSTRIPE)])
  plsc.subcore_barrier()

  for a in range(8):
    def body(b, carry, a=a):
      batch = s * TPB + b
      pltpu.sync_copy(ei_ref.at[a, c, batch], idx_v)
      pltpu.sync_copy(
          ones_v,
          hist.at[a].at[plsc.Indices(idx_v, ignored_value=-1)],
          add=True)
      return carry
    lax.fori_loop(0, TPB, body, 0)
  plsc.subcore_barrier()

  for a in range(8):
    pltpu.sync_copy(hist.at[a, pl.ds(s * DSTRIPE, DSTRIPE)],
                    deg_ref.at[a, c, pl.ds(s * DSTRIPE, DSTRIPE)])


def _make_deg_kernel():
  return pl.kernel(
      _deg_body,
      out_type=jax.ShapeDtypeStruct((8, 2, NPAD), jnp.float32),
      mesh=_sc_mesh(),
      scratch_types=[
          pltpu.VMEM_SHARED((8, NPAD), jnp.float32),
          pltpu.VMEM((BATCH,), jnp.float32),
          pltpu.VMEM((BATCH,), jnp.int32),
          pltpu.VMEM((DSTRIPE,), jnp.float32),
      ],
      compiler_params=pltpu.CompilerParams(use_tc_tiling_on_sc=False),
  )


def _agg_body(hA_ref, hB_ref, ei_ref, z_ref, out0_ref, out1_ref,
              acc, zbuf_v, sraw_v, draw_v, src_v, dst_v, rows_v, sem):
  c = lax.axis_index("c")
  s = lax.axis_index("s")
  pltpu.sync_copy(z_ref, zbuf_v)

  for e in range(2):
    h_ref = hA_ref if e == 0 else hB_ref
    out_ref = out0_ref if e == 0 else out1_ref
    pltpu.sync_copy(ei_ref.at[e, 0, pl.ds(s * EPT, EPT)], sraw_v)
    pltpu.sync_copy(ei_ref.at[e, 1, pl.ds(s * EPT, EPT)], draw_v)
    for k in range(NCHUNK // NCORE):
      base = (c * (NCHUNK // NCORE) + k) * CHUNK
      pltpu.sync_copy(zbuf_v, acc.at[pl.ds(s * CSTRIPE, CSTRIPE)])
      plsc.subcore_barrier()

      def body(b, carry, h_ref=h_ref, base=base):
        neg1 = jnp.full((16,), -1, jnp.int32)
        for i in range(BATCH // 16):
          sl = pl.ds(b * BATCH + i * 16, 16)
          ol = pl.ds(i * 16, 16)
          sv = sraw_v[sl]
          t = draw_v[sl] - base
          ok = (t >= 0) & (t < CHUNK)
          src_v[ol] = jnp.where(ok, sv, neg1)
          dst_v[ol] = jnp.where(ok, t, neg1)
        pltpu.async_copy(
            h_ref.at[plsc.Indices(src_v, ignored_value=-1)],
            rows_v, sem).wait()
        pltpu.sync_copy(
            rows_v,
            acc.at[plsc.Indices(dst_v, ignored_value=-1)],
            add=True)
        return carry
      lax.fori_loop(0, TPB, body, 0)
      plsc.subcore_barrier()

      pltpu.sync_copy(
          acc.at[pl.ds(s * CSTRIPE, CSTRIPE)],
          out_ref.at[pl.ds(base + s * CSTRIPE, CSTRIPE)])
      plsc.subcore_barrier()


def _make_agg_kernel():
  return pl.kernel(
      _agg_body,
      out_type=(jax.ShapeDtypeStruct((NPAD, D), jnp.float32),
                jax.ShapeDtypeStruct((NPAD, D), jnp.float32)),
      mesh=_sc_mesh(),
      scratch_types=[
          pltpu.VMEM_SHARED((CHUNK, D), jnp.float32),
          pltpu.VMEM((CSTRIPE, D), jnp.float32),
          pltpu.VMEM((EPT,), jnp.int32),
          pltpu.VMEM((EPT,), jnp.int32),
          pltpu.VMEM((BATCH,), jnp.int32),
          pltpu.VMEM((BATCH,), jnp.int32),
          pltpu.VMEM((BATCH, D), jnp.float32),
          pltpu.SemaphoreType.DMA,
      ],
  )


def _dot_t(x, w):
  return lax.dot_general(x, w, (((1,), (1,)), ((), ())),
                         preferred_element_type=jnp.float32)


def _dot(x, w):
  return lax.dot_general(x, w, (((1,), (0,)), ((), ())),
                         preferred_element_type=jnp.float32)


def _stage_a_body(x_ref, wf_ref, no0_ref, no1_ref,
                  h0_ref, ha_ref, hb_ref):
  h0 = jnp.tanh(_dot_t(x_ref[...], wf_ref[...]))
  h0_ref[...] = h0
  ha_ref[...] = h0 * no0_ref[...]
  hb_ref[...] = h0 * no1_ref[...]


def _attn_combine(o0, o1, wq, bq, q):
  s0 = _dot(jnp.tanh(_dot(o0, wq) + bq), q)
  s1 = _dot(jnp.tanh(_dot(o1, wq) + bq), q)
  m = jnp.maximum(s0, s1)
  e0 = jnp.exp(s0 - m)
  e1 = jnp.exp(s1 - m)
  inv = 1.0 / (e0 + e1)
  return (e0 * inv) * o0 + (e1 * inv) * o1


def _stage_b_body(agg0_ref, agg1_ref, ni0_ref, ni1_ref, no0_ref, no1_ref,
                  w00_ref, w01_ref, w10_ref, w11_ref,
                  b00_ref, b01_ref, b10_ref, b11_ref,
                  wq0_ref, wq1_ref, bq0_ref, bq1_ref, q0_ref, q1_ref,
                  h1_ref, ma_ref, mb_ref):
  g0 = agg0_ref[...] * ni0_ref[...]
  g1 = agg1_ref[...] * ni1_ref[...]
  o00 = _dot_t(g0, w00_ref[...]) + b00_ref[...]
  o01 = _dot_t(g1, w01_ref[...]) + b01_ref[...]
  m0 = _attn_combine(o00, o01, wq0_ref[...], bq0_ref[...], q0_ref[...])
  o10 = _dot_t(g0, w10_ref[...]) + b10_ref[...]
  o11 = _dot_t(g1, w11_ref[...]) + b11_ref[...]
  m1 = _attn_combine(o10, o11, wq1_ref[...], bq1_ref[...], q1_ref[...])
  h1_ref[...] = m1
  ma_ref[...] = m0 * no0_ref[...]
  mb_ref[...] = m0 * no1_ref[...]


def _stage_c_body(agg0_ref, agg1_ref, ni0_ref, ni1_ref, h0_ref, h1_ref,
                  w10_ref, w11_ref, b10_ref, b11_ref,
                  wq1_ref, bq1_ref, q1_ref,
                  wc0_ref, wc1_ref, wc2_ref, out_ref):
  g0 = agg0_ref[...] * ni0_ref[...]
  g1 = agg1_ref[...] * ni1_ref[...]
  o10 = _dot_t(g0, w10_ref[...]) + b10_ref[...]
  o11 = _dot_t(g1, w11_ref[...]) + b11_ref[...]
  h2 = _attn_combine(o10, o11, wq1_ref[...], bq1_ref[...], q1_ref[...])
  out_ref[...] = (_dot_t(h0_ref[...], wc0_ref[...])
                  + _dot_t(h1_ref[...], wc1_ref[...])
                  + _dot_t(h2, wc2_ref[...]))


def _row_spec():
  return pl.BlockSpec((RB, D), lambda i: (i, 0))


def _col_spec():
  return pl.BlockSpec((RB, 1), lambda i: (i, 0))


def _full_spec(shape):
  return pl.BlockSpec(shape, lambda i: tuple(0 for _ in shape))


def _stage_a(x, wf, no0, no1):
  return pl.pallas_call(
      _stage_a_body,
      grid=(NRB,),
      in_specs=[_row_spec(), _full_spec((D, D)), _col_spec(), _col_spec()],
      out_specs=[_row_spec(), _row_spec(), _row_spec()],
      out_shape=[jax.ShapeDtypeStruct((NPAD, D), jnp.float32)] * 3,
  )(x, wf, no0, no1)


def _stage_b(agg0, agg1, ni0, ni1, no0, no1, w, b, wq, bq, q):
  return pl.pallas_call(
      _stage_b_body,
      grid=(NRB,),
      in_specs=([_row_spec(), _row_spec()] + [_col_spec()] * 4
                + [_full_spec((D, D))] * 4
                + [_full_spec((1, D))] * 4
                + [_full_spec((D, 64))] * 2
                + [_full_spec((1, 64))] * 2
                + [_full_spec((64, 1))] * 2),
      out_specs=[_row_spec()] * 3,
      out_shape=[jax.ShapeDtypeStruct((NPAD, D), jnp.float32)] * 3,
  )(agg0, agg1, ni0, ni1, no0, no1,
    w[0][0], w[0][1], w[1][0], w[1][1],
    b[0][0], b[0][1], b[1][0], b[1][1],
    wq[0], wq[1], bq[0], bq[1], q[0], q[1])


def _stage_c(agg0, agg1, ni0, ni1, h0, h1, w, b, wq1, bq1, q1, wc):
  return pl.pallas_call(
      _stage_c_body,
      grid=(NRB,),
      in_specs=([_row_spec(), _row_spec()] + [_col_spec()] * 2
                + [_row_spec(), _row_spec()]
                + [_full_spec((D, D))] * 2
                + [_full_spec((1, D))] * 2
                + [_full_spec((D, 64)), _full_spec((1, 64)),
                   _full_spec((64, 1))]
                + [_full_spec((D, D))] * 3),
      out_specs=pl.BlockSpec((RB, D), lambda i: (i, 0)),
      out_shape=jax.ShapeDtypeStruct((N, D), jnp.float32),
  )(agg0, agg1, ni0, ni1, h0, h1,
    w[1][0], w[1][1], b[1][0], b[1][1],
    wq1, bq1, q1, wc[0], wc[1], wc[2])


def _pad_edges(ei):
  pad = jnp.full((2, EP - E), -1, jnp.int32)
  return jnp.concatenate([ei.astype(jnp.int32), pad], axis=1).reshape(
      2, NB, BATCH)


def _prep_params(p):
  return dict(
      wf=p['Wf'],
      w=p['W'],
      b=[[p['b'][l][r].reshape(1, D) for r in range(2)] for l in range(2)],
      wq=p['Wq'],
      bq=[p['bq'][l].reshape(1, 64) for l in range(2)],
      q=[p['q'][l].reshape(64, 1) for l in range(2)],
      wc=[p['Wc'][:, i * D:(i + 1) * D] for i in range(3)],
  )


def kernel(x_attr, x_stru, ei_attr_pos0, ei_attr_pos1, ei_attr_neg0,
           ei_attr_neg1, ei_stru_pos0, ei_stru_pos1, ei_stru_neg0,
           ei_stru_neg1, params_pos, params_neg):
  edge_arrays = [ei_attr_pos0, ei_attr_pos1, ei_attr_neg0, ei_attr_neg1,
                 ei_stru_pos0, ei_stru_pos1, ei_stru_neg0, ei_stru_neg1]
  ei_pad = [_pad_edges(e) for e in edge_arrays]
  ei_all = jnp.stack(ei_pad)

  degs = _make_deg_kernel()(ei_all)
  norms = jnp.where(degs > 0, degs, 1.0) ** -0.5

  zeros_hbm = jnp.zeros((CSTRIPE, D), jnp.float32)
  agg_kernel = _make_agg_kernel()

  pp = _prep_params(params_pos)
  pn = _prep_params(params_neg)

  branches = [
      (x_attr, 0, 1, pp),
      (x_stru, 4, 5, pp),
      (x_attr, 2, 3, pn),
      (x_stru, 6, 7, pn),
  ]

  outs = []
  for x, a0, a1, p in branches:
    no0 = norms[a0, 0][:, None]
    no1 = norms[a1, 0][:, None]
    ni0 = norms[a0, 1][:, None]
    ni1 = norms[a1, 1][:, None]
    ei2 = jnp.stack([ei_pad[a0], ei_pad[a1]]).reshape(2, 2, EP)

    h0, ha, hb = _stage_a(x, p['wf'], no0, no1)
    agg0, agg1 = agg_kernel(ha, hb, ei2, zeros_hbm)
    h1, ma, mb = _stage_b(agg0, agg1, ni0, ni1, no0, no1,
                          p['w'], p['b'], p['wq'], p['bq'], p['q'])
    aggm0, aggm1 = agg_kernel(ma, mb, ei2, zeros_hbm)
    out = _stage_c(aggm0, aggm1, ni0, ni1, h0, h1,
                   p['w'], p['b'], p['wq'][1], p['bq'][1], p['q'][1],
                   p['wc'])
    outs.append(out)

  return tuple(outs)


# speedup vs baseline: 2.7099x; 1.1812x over previous
import functools

import jax
import jax.numpy as jnp
from jax import lax
from jax.experimental import pallas as pl
from jax.experimental.pallas import tpu as pltpu
from jax.experimental.pallas import tpu_sc as plsc

N = 50000
D = 128
E = 50000
BATCH = 128
NB = 400
EP = NB * BATCH
NSUB = 16
NCORE = 2
TPB = NB // NSUB
NPAD = 50176
DSTRIPE = NPAD // NSUB
NCHUNK = 8
CHUNK = NPAD // NCHUNK
KPC = NCHUNK // NCORE
CSTRIPE = CHUNK // NSUB
ZROWS = 56
EPT = EP // NSUB
BUFS = 3
RB = 512
NRB = NPAD // RB


def _sc_mesh():
  return plsc.VectorSubcoreMesh(
      core_axis_name="c", subcore_axis_name="s",
      num_cores=NCORE, num_subcores=NSUB)


def _deg_body(ei_ref, deg_ref, hist, ones_v, idx_v, zrow_v):
  c = lax.axis_index("c")
  s = lax.axis_index("s")
  one16 = jnp.ones((16,), jnp.float32)
  zero16 = jnp.zeros((16,), jnp.float32)
  for i in range(BATCH // 16):
    ones_v[pl.ds(i * 16, 16)] = one16

  def zinit(i, carry):
    zrow_v[pl.ds(i * 16, 16)] = zero16
    return carry
  lax.fori_loop(0, DSTRIPE // 16, zinit, 0)

  for a in range(8):
    pltpu.sync_copy(zrow_v,
                    hist.at[a, pl.ds(s * DSTRIPE, DSTRIPE)])
  plsc.subcore_barrier()

  for a in range(8):
    def body(b, carry, a=a):
      batch = s * TPB + b
      pltpu.sync_copy(ei_ref.at[a, c, batch], idx_v)
      pltpu.sync_copy(
          ones_v,
          hist.at[a].at[plsc.Indices(idx_v, ignored_value=-1)],
          add=True)
      return carry
    lax.fori_loop(0, TPB, body, 0)
  plsc.subcore_barrier()

  for a in range(8):
    pltpu.sync_copy(hist.at[a, pl.ds(s * DSTRIPE, DSTRIPE)],
                    deg_ref.at[a, c, pl.ds(s * DSTRIPE, DSTRIPE)])


def _make_deg_kernel():
  return pl.kernel(
      _deg_body,
      out_type=jax.ShapeDtypeStruct((8, 2, NPAD), jnp.float32),
      mesh=_sc_mesh(),
      scratch_types=[
          pltpu.VMEM_SHARED((8, NPAD), jnp.float32),
          pltpu.VMEM((BATCH,), jnp.float32),
          pltpu.VMEM((BATCH,), jnp.int32),
          pltpu.VMEM((DSTRIPE,), jnp.float32),
      ],
      compiler_params=pltpu.CompilerParams(use_tc_tiling_on_sc=False),
  )


def _agg_body(hA_ref, hB_ref, es0_ref, ed0_ref, es1_ref, ed1_ref,
              z_ref, out0_ref, out1_ref,
              acc, zbuf_v, sb0_v, sb1_v, sb2_v, db0_v, db1_v, db2_v,
              src0_v, src1_v, src2_v, dst0_v, dst1_v, dst2_v,
              row0_v, row1_v, row2_v, semG, semS):
  srcs_v = [src0_v, src1_v, src2_v]
  dsts_v = [dst0_v, dst1_v, dst2_v]
  rows_v = [row0_v, row1_v, row2_v]
  sbat_v = [sb0_v, sb1_v, sb2_v]
  dbat_v = [db0_v, db1_v, db2_v]
  c = lax.axis_index("c")
  s = lax.axis_index("s")
  neg1 = jnp.full((16,), -1, jnp.int32)
  pltpu.sync_copy(z_ref, zbuf_v)

  for e in range(2):
    h_ref = hA_ref if e == 0 else hB_ref
    out_ref = out0_ref if e == 0 else out1_ref
    es_ref = es0_ref if e == 0 else es1_ref
    ed_ref = ed0_ref if e == 0 else ed1_ref

    for k in range(KPC):
      base = (c * KPC + k) * CHUNK
      for z in range(CSTRIPE // ZROWS):
        pltpu.sync_copy(
            zbuf_v, acc.at[pl.ds(s * CSTRIPE + z * ZROWS, ZROWS)])
      plsc.subcore_barrier()

      def prep(b, u, base=base, es_ref=es_ref, ed_ref=ed_ref):
        goff = (s * TPB + b) * BATCH
        pltpu.sync_copy(es_ref.at[pl.ds(goff, BATCH)], sbat_v[u])
        pltpu.sync_copy(ed_ref.at[pl.ds(goff, BATCH)], dbat_v[u])
        for i in range(BATCH // 16):
          ol = pl.ds(i * 16, 16)
          sv = sbat_v[u][ol]
          t = dbat_v[u][ol] - base
          ok = (t >= 0) & (t < CHUNK)
          srcs_v[u][ol] = jnp.where(ok, sv, neg1)
          dsts_v[u][ol] = jnp.where(ok, t, neg1)

      def gather(u, h_ref=h_ref):
        return pltpu.make_async_copy(
            h_ref.at[plsc.Indices(srcs_v[u], ignored_value=-1)],
            rows_v[u], semG[u])

      def scat(u):
        return pltpu.make_async_copy(
            rows_v[u],
            acc.at[plsc.Indices(dsts_v[u], ignored_value=-1)],
            semS[u])

      def step(j, carry):
        for u in range(BUFS):
          b = j * BUFS + u

          @pl.when(b < TPB)
          def _issue(u=u, b=b):
            @pl.when(b >= BUFS)
            def _drain_prev():
              scat(u).wait()
            prep(b, u)
            gather(u).start()

        for u in range(BUFS):
          b = j * BUFS + u

          @pl.when(b < TPB)
          def _finish(u=u):
            gather(u).wait()
            scat(u).start(add=True)
        return carry
      lax.fori_loop(0, (TPB + BUFS - 1) // BUFS, step, 0)
      for u in range(BUFS):
        scat(u).wait()
      plsc.subcore_barrier()

      pltpu.sync_copy(
          acc.at[pl.ds(s * CSTRIPE, CSTRIPE)],
          out_ref.at[pl.ds(base + s * CSTRIPE, CSTRIPE)])
      plsc.subcore_barrier()


def _make_agg_kernel():
  return pl.kernel(
      _agg_body,
      out_type=(jax.ShapeDtypeStruct((NPAD, D), jnp.float32),
                jax.ShapeDtypeStruct((NPAD, D), jnp.float32)),
      mesh=_sc_mesh(),
      scratch_types=[
          pltpu.VMEM_SHARED((CHUNK, D), jnp.float32),
          pltpu.VMEM((ZROWS, D), jnp.float32),
          pltpu.VMEM((BATCH,), jnp.int32),
          pltpu.VMEM((BATCH,), jnp.int32),
          pltpu.VMEM((BATCH,), jnp.int32),
          pltpu.VMEM((BATCH,), jnp.int32),
          pltpu.VMEM((BATCH,), jnp.int32),
          pltpu.VMEM((BATCH,), jnp.int32),
          pltpu.VMEM((BATCH,), jnp.int32),
          pltpu.VMEM((BATCH,), jnp.int32),
          pltpu.VMEM((BATCH,), jnp.int32),
          pltpu.VMEM((BATCH,), jnp.int32),
          pltpu.VMEM((BATCH,), jnp.int32),
          pltpu.VMEM((BATCH,), jnp.int32),
          pltpu.VMEM((BATCH, D), jnp.float32),
          pltpu.VMEM((BATCH, D), jnp.float32),
          pltpu.VMEM((BATCH, D), jnp.float32),
          [pltpu.SemaphoreType.DMA] * BUFS,
          [pltpu.SemaphoreType.DMA] * BUFS,
      ],
  )


def _dot_t(x, w):
  return lax.dot_general(x, w, (((1,), (1,)), ((), ())),
                         preferred_element_type=jnp.float32)


def _dot(x, w):
  return lax.dot_general(x, w, (((1,), (0,)), ((), ())),
                         preferred_element_type=jnp.float32)


def _stage_a_body(x_ref, wf_ref, no0_ref, no1_ref,
                  h0_ref, ha_ref, hb_ref):
  h0 = jnp.tanh(_dot_t(x_ref[...], wf_ref[...]))
  h0_ref[...] = h0
  ha_ref[...] = h0 * no0_ref[...]
  hb_ref[...] = h0 * no1_ref[...]


def _attn_combine(o0, o1, wq, bq, q):
  s0 = _dot(jnp.tanh(_dot(o0, wq) + bq), q)
  s1 = _dot(jnp.tanh(_dot(o1, wq) + bq), q)
  m = jnp.maximum(s0, s1)
  e0 = jnp.exp(s0 - m)
  e1 = jnp.exp(s1 - m)
  inv = 1.0 / (e0 + e1)
  return (e0 * inv) * o0 + (e1 * inv) * o1


def _stage_b_body(agg0_ref, agg1_ref, ni0_ref, ni1_ref, no0_ref, no1_ref,
                  w00_ref, w01_ref, w10_ref, w11_ref,
                  b00_ref, b01_ref, b10_ref, b11_ref,
                  wq0_ref, wq1_ref, bq0_ref, bq1_ref, q0_ref, q1_ref,
                  h1_ref, ma_ref, mb_ref):
  g0 = agg0_ref[...] * ni0_ref[...]
  g1 = agg1_ref[...] * ni1_ref[...]
  o00 = _dot_t(g0, w00_ref[...]) + b00_ref[...]
  o01 = _dot_t(g1, w01_ref[...]) + b01_ref[...]
  m0 = _attn_combine(o00, o01, wq0_ref[...], bq0_ref[...], q0_ref[...])
  o10 = _dot_t(g0, w10_ref[...]) + b10_ref[...]
  o11 = _dot_t(g1, w11_ref[...]) + b11_ref[...]
  m1 = _attn_combine(o10, o11, wq1_ref[...], bq1_ref[...], q1_ref[...])
  h1_ref[...] = m1
  ma_ref[...] = m0 * no0_ref[...]
  mb_ref[...] = m0 * no1_ref[...]


def _stage_c_body(agg0_ref, agg1_ref, ni0_ref, ni1_ref, h0_ref, h1_ref,
                  w10_ref, w11_ref, b10_ref, b11_ref,
                  wq1_ref, bq1_ref, q1_ref,
                  wc0_ref, wc1_ref, wc2_ref, out_ref):
  g0 = agg0_ref[...] * ni0_ref[...]
  g1 = agg1_ref[...] * ni1_ref[...]
  o10 = _dot_t(g0, w10_ref[...]) + b10_ref[...]
  o11 = _dot_t(g1, w11_ref[...]) + b11_ref[...]
  h2 = _attn_combine(o10, o11, wq1_ref[...], bq1_ref[...], q1_ref[...])
  out_ref[...] = (_dot_t(h0_ref[...], wc0_ref[...])
                  + _dot_t(h1_ref[...], wc1_ref[...])
                  + _dot_t(h2, wc2_ref[...]))


def _row_spec():
  return pl.BlockSpec((RB, D), lambda i: (i, 0))


def _col_spec():
  return pl.BlockSpec((RB, 1), lambda i: (i, 0))


def _full_spec(shape):
  return pl.BlockSpec(shape, lambda i: tuple(0 for _ in shape))


def _stage_a(x, wf, no0, no1):
  return pl.pallas_call(
      _stage_a_body,
      grid=(NRB,),
      in_specs=[_row_spec(), _full_spec((D, D)), _col_spec(), _col_spec()],
      out_specs=[_row_spec(), _row_spec(), _row_spec()],
      out_shape=[jax.ShapeDtypeStruct((NPAD, D), jnp.float32)] * 3,
  )(x, wf, no0, no1)


def _stage_b(agg0, agg1, ni0, ni1, no0, no1, w, b, wq, bq, q):
  return pl.pallas_call(
      _stage_b_body,
      grid=(NRB,),
      in_specs=([_row_spec(), _row_spec()] + [_col_spec()] * 4
                + [_full_spec((D, D))] * 4
                + [_full_spec((1, D))] * 4
                + [_full_spec((D, 64))] * 2
                + [_full_spec((1, 64))] * 2
                + [_full_spec((64, 1))] * 2),
      out_specs=[_row_spec()] * 3,
      out_shape=[jax.ShapeDtypeStruct((NPAD, D), jnp.float32)] * 3,
  )(agg0, agg1, ni0, ni1, no0, no1,
    w[0][0], w[0][1], w[1][0], w[1][1],
    b[0][0], b[0][1], b[1][0], b[1][1],
    wq[0], wq[1], bq[0], bq[1], q[0], q[1])


def _stage_c(agg0, agg1, ni0, ni1, h0, h1, w, b, wq1, bq1, q1, wc):
  return pl.pallas_call(
      _stage_c_body,
      grid=(NRB,),
      in_specs=([_row_spec(), _row_spec()] + [_col_spec()] * 2
                + [_row_spec(), _row_spec()]
                + [_full_spec((D, D))] * 2
                + [_full_spec((1, D))] * 2
                + [_full_spec((D, 64)), _full_spec((1, 64)),
                   _full_spec((64, 1))]
                + [_full_spec((D, D))] * 3),
      out_specs=pl.BlockSpec((RB, D), lambda i: (i, 0)),
      out_shape=jax.ShapeDtypeStruct((N, D), jnp.float32),
  )(agg0, agg1, ni0, ni1, h0, h1,
    w[1][0], w[1][1], b[1][0], b[1][1],
    wq1, bq1, q1, wc[0], wc[1], wc[2])


def _pad_edges(ei):
  pad = jnp.full((2, EP - E), -1, jnp.int32)
  return jnp.concatenate([ei.astype(jnp.int32), pad], axis=1).reshape(
      2, NB, BATCH)


def _prep_params(p):
  return dict(
      wf=p['Wf'],
      w=p['W'],
      b=[[p['b'][l][r].reshape(1, D) for r in range(2)] for l in range(2)],
      wq=p['Wq'],
      bq=[p['bq'][l].reshape(1, 64) for l in range(2)],
      q=[p['q'][l].reshape(64, 1) for l in range(2)],
      wc=[p['Wc'][:, i * D:(i + 1) * D] for i in range(3)],
  )


def kernel(x_attr, x_stru, ei_attr_pos0, ei_attr_pos1, ei_attr_neg0,
           ei_attr_neg1, ei_stru_pos0, ei_stru_pos1, ei_stru_neg0,
           ei_stru_neg1, params_pos, params_neg):
  edge_arrays = [ei_attr_pos0, ei_attr_pos1, ei_attr_neg0, ei_attr_neg1,
                 ei_stru_pos0, ei_stru_pos1, ei_stru_neg0, ei_stru_neg1]
  ei_pad = [_pad_edges(e) for e in edge_arrays]
  ei_all = jnp.stack(ei_pad)

  degs = _make_deg_kernel()(ei_all)
  norms = jnp.where(degs > 0, degs, 1.0) ** -0.5

  zeros_hbm = jnp.zeros((ZROWS, D), jnp.float32)
  agg_kernel = _make_agg_kernel()

  pp = _prep_params(params_pos)
  pn = _prep_params(params_neg)

  branches = [
      (x_attr, 0, 1, pp),
      (x_stru, 4, 5, pp),
      (x_attr, 2, 3, pn),
      (x_stru, 6, 7, pn),
  ]

  outs = []
  for x, a0, a1, p in branches:
    no0 = norms[a0, 0][:, None]
    no1 = norms[a1, 0][:, None]
    ni0 = norms[a0, 1][:, None]
    ni1 = norms[a1, 1][:, None]
    es0 = ei_pad[a0][0].reshape(EP)
    ed0 = ei_pad[a0][1].reshape(EP)
    es1 = ei_pad[a1][0].reshape(EP)
    ed1 = ei_pad[a1][1].reshape(EP)

    h0, ha, hb = _stage_a(x, p['wf'], no0, no1)
    agg0, agg1 = agg_kernel(ha, hb, es0, ed0, es1, ed1, zeros_hbm)
    h1, ma, mb = _stage_b(agg0, agg1, ni0, ni1, no0, no1,
                          p['w'], p['b'], p['wq'], p['bq'], p['q'])
    aggm0, aggm1 = agg_kernel(ma, mb, es0, ed0, es1, ed1, zeros_hbm)
    out = _stage_c(aggm0, aggm1, ni0, ni1, h0, h1,
                   p['w'], p['b'], p['wq'][1], p['bq'][1], p['q'][1],
                   p['wc'])
    outs.append(out)

  return tuple(outs)
